# baseline (speedup 1.0000x reference)
import jax
import jax.numpy as jnp
from jax import lax
from jax.experimental import pallas as pl
from jax.experimental.pallas import tpu as pltpu
from jax.experimental.pallas import tpu_sc as plsc

BATCH = 16384
VOCAB = 1000000
DIM = 64
WIDE = 2 * DIM

NUM_CORES = 2
NUM_SUBCORES = 16
NUM_WORKERS = NUM_CORES * NUM_SUBCORES
CHUNK = 128
CHUNKS_PER_WORKER = BATCH // (NUM_WORKERS * CHUNK)

HBLK = 2048
FBLK = 2 * HBLK
RGRID = -(-VOCAB // FBLK)
PACKED_ROWS = RGRID * HBLK


def _relayout_body(a_ref, b_ref, out_ref):
    eye = jnp.eye(DIM, dtype=jnp.float32)

    def tr(x):
        return jax.lax.dot_general(
            x,
            eye,
            dimension_numbers=(((0,), (0,)), ((), ())),
            precision=jax.lax.Precision.HIGHEST,
        )

    out_ref[...] = jnp.concatenate([tr(a_ref[...]), tr(b_ref[...])], axis=1)


def _relayout(wv_t):
    return pl.pallas_call(
        _relayout_body,
        grid=(RGRID,),
        in_specs=[
            pl.BlockSpec(
                (DIM, HBLK),
                lambda i: (0, jnp.minimum(2 * i, VOCAB // HBLK)),
            ),
            pl.BlockSpec(
                (DIM, HBLK),
                lambda i: (0, jnp.minimum(2 * i + 1, VOCAB // HBLK)),
            ),
        ],
        out_specs=pl.BlockSpec((HBLK, WIDE), lambda i: (i, 0)),
        out_shape=jax.ShapeDtypeStruct((PACKED_ROWS, WIDE), jnp.float32),
    )(wv_t, wv_t)


def _sc_gather_body(table_hbm, idx_hbm, out_hbm, idx_v, rows_v, sem):
    c = lax.axis_index("c")
    s = lax.axis_index("s")
    wid = s * NUM_CORES + c
    base = wid * CHUNKS_PER_WORKER
    pltpu.sync_copy(idx_hbm.at[pl.ds(base, CHUNKS_PER_WORKER)], idx_v)
    copies = [
        pltpu.async_copy(table_hbm.at[idx_v.at[j]], rows_v.at[j], sem)
        for j in range(CHUNKS_PER_WORKER)
    ]
    for cp in copies:
        cp.wait()
    pltpu.sync_copy(rows_v, out_hbm.at[pl.ds(base, CHUNKS_PER_WORKER)])


def _sc_gather(table_packed, idx):
    mesh = plsc.VectorSubcoreMesh(core_axis_name="c", subcore_axis_name="s")
    kfn = pl.kernel(
        _sc_gather_body,
        out_type=jax.ShapeDtypeStruct(
            (NUM_WORKERS * CHUNKS_PER_WORKER, CHUNK, WIDE), jnp.float32
        ),
        mesh=mesh,
        scratch_types=[
            pltpu.VMEM((CHUNKS_PER_WORKER, CHUNK), jnp.int32),
            pltpu.VMEM((CHUNKS_PER_WORKER, CHUNK, WIDE), jnp.float32),
            pltpu.SemaphoreType.DMA,
        ],
    )
    return kfn(table_packed, idx)


GRID = 16
BLK = BATCH // GRID


def _cos_body(x_ref, sel_ref, par_ref, mask_ref, out_ref, acc_ref):
    i = pl.program_id(0)

    @pl.when(i == 0)
    def _():
        acc_ref[0] = 0.0
        acc_ref[1] = 0.0

    x = x_ref[...]
    s2 = sel_ref[...]
    p = par_ref[...]
    m = mask_ref[...]
    s = jnp.where(p > 0.5, s2[:, DIM:], s2[:, :DIM])
    num = jnp.sum(x * s, axis=1)
    n1s = jnp.sum(x * x, axis=1)
    n2s = jnp.sum(s * s, axis=1)
    denom = jnp.maximum(jnp.sqrt(n1s) * jnp.sqrt(n2s), 1e-8)
    acc_ref[0] += jnp.sum(-(num / denom) * m)
    acc_ref[1] += jnp.sum(m)

    @pl.when(i == GRID - 1)
    def _():
        out_ref[...] = jnp.full((1, 1), acc_ref[0] / acc_ref[1], jnp.float32)


def _cos_loss(x, sel, par, mask):
    return pl.pallas_call(
        _cos_body,
        grid=(GRID,),
        in_specs=[
            pl.BlockSpec((BLK, DIM), lambda i: (i, 0)),
            pl.BlockSpec((BLK, WIDE), lambda i: (i, 0)),
            pl.BlockSpec((BLK, 1), lambda i: (i, 0)),
            pl.BlockSpec((BLK,), lambda i: (i,)),
        ],
        out_specs=pl.BlockSpec((1, 1), lambda i: (0, 0)),
        out_shape=jax.ShapeDtypeStruct((1, 1), jnp.float32),
        scratch_shapes=[pltpu.SMEM((2,), jnp.float32)],
    )(x, sel, par, mask)


def kernel(logits, target, mask, word_vectors):
    table_packed = _relayout(word_vectors.T)
    q = (target // FBLK) * HBLK + (target % HBLK)
    k = (target % FBLK) // HBLK
    idx = q.reshape(NUM_WORKERS * CHUNKS_PER_WORKER, CHUNK)
    par = k.astype(jnp.float32).reshape(BATCH, 1)
    sel = _sc_gather(table_packed, idx).reshape(BATCH, WIDE)
    out = _cos_loss(logits, sel, par, mask)
    return out[0, 0]


# speedup vs baseline: 2.0185x; 2.0185x over previous
import jax
import jax.numpy as jnp
from jax import lax
from jax.experimental import pallas as pl
from jax.experimental.pallas import tpu as pltpu
from jax.experimental.pallas import tpu_sc as plsc

BATCH = 16384
VOCAB = 1000000
DIM = 64
WIDE = 2 * DIM

NUM_CORES = 2
NUM_SUBCORES = 16
NUM_WORKERS = NUM_CORES * NUM_SUBCORES
CHUNK = 128
CHUNKS_PER_WORKER = BATCH // (NUM_WORKERS * CHUNK)

HBLK = 4096
FBLK = 2 * HBLK
RGRID = -(-VOCAB // FBLK)
PACKED_ROWS = RGRID * HBLK


def _relayout_body(a_ref, b_ref, out_ref):
    out_ref[...] = jnp.concatenate(
        [jnp.transpose(a_ref[...]), jnp.transpose(b_ref[...])], axis=1
    )


def _relayout(wv_t):
    return pl.pallas_call(
        _relayout_body,
        grid=(RGRID,),
        in_specs=[
            pl.BlockSpec(
                (DIM, HBLK),
                lambda i: (0, jnp.minimum(2 * i, VOCAB // HBLK)),
            ),
            pl.BlockSpec(
                (DIM, HBLK),
                lambda i: (0, jnp.minimum(2 * i + 1, VOCAB // HBLK)),
            ),
        ],
        out_specs=pl.BlockSpec((HBLK, WIDE), lambda i: (i, 0)),
        out_shape=jax.ShapeDtypeStruct((PACKED_ROWS, WIDE), jnp.float32),
    )(wv_t, wv_t)


def _sc_gather_body(table_hbm, idx_hbm, out_hbm, idx_v, rows_v, sem):
    c = lax.axis_index("c")
    s = lax.axis_index("s")
    wid = s * NUM_CORES + c
    base = wid * CHUNKS_PER_WORKER
    pltpu.sync_copy(idx_hbm.at[pl.ds(base, CHUNKS_PER_WORKER)], idx_v)
    copies = [
        pltpu.async_copy(table_hbm.at[idx_v.at[j]], rows_v.at[j], sem)
        for j in range(CHUNKS_PER_WORKER)
    ]
    for cp in copies:
        cp.wait()
    pltpu.sync_copy(rows_v, out_hbm.at[pl.ds(base, CHUNKS_PER_WORKER)])


def _sc_gather(table_packed, idx):
    mesh = plsc.VectorSubcoreMesh(core_axis_name="c", subcore_axis_name="s")
    kfn = pl.kernel(
        _sc_gather_body,
        out_type=jax.ShapeDtypeStruct(
            (NUM_WORKERS * CHUNKS_PER_WORKER, CHUNK, WIDE), jnp.float32
        ),
        mesh=mesh,
        scratch_types=[
            pltpu.VMEM((CHUNKS_PER_WORKER, CHUNK), jnp.int32),
            pltpu.VMEM((CHUNKS_PER_WORKER, CHUNK, WIDE), jnp.float32),
            pltpu.SemaphoreType.DMA,
        ],
    )
    return kfn(table_packed, idx)


GRID = 8
BLK = BATCH // GRID


def _cos_body(xt_ref, sel_ref, par_ref, mask_ref, out_ref, acc_ref):
    i = pl.program_id(0)

    @pl.when(i == 0)
    def _():
        acc_ref[0] = 0.0
        acc_ref[1] = 0.0

    x = xt_ref[...]
    s2 = sel_ref[...]
    p = par_ref[...]
    m = mask_ref[...]
    eye = jnp.eye(WIDE, dtype=jnp.float32)
    ts = jax.lax.dot_general(
        eye,
        s2,
        dimension_numbers=(((1,), (1,)), ((), ())),
        precision=jax.lax.Precision.HIGHEST,
    )
    s = jnp.where(p[None, :] > 0.5, ts[DIM:, :], ts[:DIM, :])
    num = jnp.sum(x * s, axis=0)
    n1s = jnp.sum(x * x, axis=0)
    n2s = jnp.sum(s * s, axis=0)
    denom = jnp.maximum(jnp.sqrt(n1s) * jnp.sqrt(n2s), 1e-8)
    acc_ref[0] += jnp.sum(-(num / denom) * m)
    acc_ref[1] += jnp.sum(m)

    @pl.when(i == GRID - 1)
    def _():
        out_ref[...] = jnp.full((1, 1), acc_ref[0] / acc_ref[1], jnp.float32)


def _cos_loss(x_t, sel, par, mask):
    return pl.pallas_call(
        _cos_body,
        grid=(GRID,),
        in_specs=[
            pl.BlockSpec((DIM, BLK), lambda i: (0, i)),
            pl.BlockSpec((BLK, WIDE), lambda i: (i, 0)),
            pl.BlockSpec((BLK,), lambda i: (i,)),
            pl.BlockSpec((BLK,), lambda i: (i,)),
        ],
        out_specs=pl.BlockSpec((1, 1), lambda i: (0, 0)),
        out_shape=jax.ShapeDtypeStruct((1, 1), jnp.float32),
        scratch_shapes=[pltpu.SMEM((2,), jnp.float32)],
    )(x_t, sel, par, mask)


def kernel(logits, target, mask, word_vectors):
    table_packed = _relayout(word_vectors.T)
    q = (target // FBLK) * HBLK + (target % HBLK)
    k = (target % FBLK) // HBLK
    idx = q.reshape(NUM_WORKERS * CHUNKS_PER_WORKER, CHUNK)
    par = k.astype(jnp.float32)
    sel = _sc_gather(table_packed, idx).reshape(BATCH, WIDE)
    out = _cos_loss(logits.T, sel, par, mask)
    return out[0, 0]


# speedup vs baseline: 2.2719x; 1.1256x over previous
import jax
import jax.numpy as jnp
from jax import lax
from jax.experimental import pallas as pl
from jax.experimental.pallas import tpu as pltpu
from jax.experimental.pallas import tpu_sc as plsc

BATCH = 16384
VOCAB = 1000000
DIM = 64
WIDE = 2 * DIM

NUM_CORES = 2
NUM_SUBCORES = 16
NUM_WORKERS = NUM_CORES * NUM_SUBCORES
CHUNK = 128
CHUNKS_PER_WORKER = BATCH // (NUM_WORKERS * CHUNK)

HBLK = 8192
FBLK = 2 * HBLK
RGRID = -(-VOCAB // FBLK)
PACKED_ROWS = RGRID * HBLK


def _relayout_body(a_ref, b_ref, out_ref):
    out_ref[:, :DIM] = jnp.transpose(a_ref[...])
    out_ref[:, DIM:] = jnp.transpose(b_ref[...])


def _relayout(wv_t):
    return pl.pallas_call(
        _relayout_body,
        grid=(RGRID,),
        in_specs=[
            pl.BlockSpec(
                (DIM, HBLK),
                lambda i: (0, jnp.minimum(2 * i, VOCAB // HBLK)),
            ),
            pl.BlockSpec(
                (DIM, HBLK),
                lambda i: (0, jnp.minimum(2 * i + 1, VOCAB // HBLK)),
            ),
        ],
        out_specs=pl.BlockSpec((HBLK, WIDE), lambda i: (i, 0)),
        out_shape=jax.ShapeDtypeStruct((PACKED_ROWS, WIDE), jnp.float32),
    )(wv_t, wv_t)


def _sc_gather_body(table_hbm, idx_hbm, out_hbm, idx_v, rows_v, sem):
    c = lax.axis_index("c")
    s = lax.axis_index("s")
    wid = s * NUM_CORES + c
    base = wid * CHUNKS_PER_WORKER
    pltpu.sync_copy(idx_hbm.at[pl.ds(base, CHUNKS_PER_WORKER)], idx_v)
    copies = [
        pltpu.async_copy(table_hbm.at[idx_v.at[j]], rows_v.at[j], sem)
        for j in range(CHUNKS_PER_WORKER)
    ]
    for cp in copies:
        cp.wait()
    pltpu.sync_copy(rows_v, out_hbm.at[pl.ds(base, CHUNKS_PER_WORKER)])


def _sc_gather(table_packed, idx):
    mesh = plsc.VectorSubcoreMesh(core_axis_name="c", subcore_axis_name="s")
    kfn = pl.kernel(
        _sc_gather_body,
        out_type=jax.ShapeDtypeStruct(
            (NUM_WORKERS * CHUNKS_PER_WORKER, CHUNK, WIDE), jnp.float32
        ),
        mesh=mesh,
        scratch_types=[
            pltpu.VMEM((CHUNKS_PER_WORKER, CHUNK), jnp.int32),
            pltpu.VMEM((CHUNKS_PER_WORKER, CHUNK, WIDE), jnp.float32),
            pltpu.SemaphoreType.DMA,
        ],
    )
    return kfn(table_packed, idx)


GRID = 8
BLK = BATCH // GRID


def _cos_body(xt_ref, sel_ref, par_ref, mask_ref, out_ref, acc_ref):
    i = pl.program_id(0)

    @pl.when(i == 0)
    def _():
        acc_ref[0] = 0.0
        acc_ref[1] = 0.0

    x = xt_ref[...]
    s2 = sel_ref[...]
    p = par_ref[...]
    m = mask_ref[...]
    eye = jnp.eye(WIDE, dtype=jnp.float32)
    ts = jax.lax.dot_general(
        eye,
        s2,
        dimension_numbers=(((1,), (1,)), ((), ())),
        precision=jax.lax.Precision.HIGHEST,
    )
    s = jnp.where(p[None, :] > 0.5, ts[DIM:, :], ts[:DIM, :])
    num = jnp.sum(x * s, axis=0)
    n1s = jnp.sum(x * x, axis=0)
    n2s = jnp.sum(s * s, axis=0)
    denom = jnp.maximum(jnp.sqrt(n1s) * jnp.sqrt(n2s), 1e-8)
    acc_ref[0] += jnp.sum(-(num / denom) * m)
    acc_ref[1] += jnp.sum(m)

    @pl.when(i == GRID - 1)
    def _():
        out_ref[...] = jnp.full((1, 1), acc_ref[0] / acc_ref[1], jnp.float32)


def _cos_loss(x_t, sel, par, mask):
    return pl.pallas_call(
        _cos_body,
        grid=(GRID,),
        in_specs=[
            pl.BlockSpec((DIM, BLK), lambda i: (0, i)),
            pl.BlockSpec((BLK, WIDE), lambda i: (i, 0)),
            pl.BlockSpec((BLK,), lambda i: (i,)),
            pl.BlockSpec((BLK,), lambda i: (i,)),
        ],
        out_specs=pl.BlockSpec((1, 1), lambda i: (0, 0)),
        out_shape=jax.ShapeDtypeStruct((1, 1), jnp.float32),
        scratch_shapes=[pltpu.SMEM((2,), jnp.float32)],
    )(x_t, sel, par, mask)


def kernel(logits, target, mask, word_vectors):
    table_packed = _relayout(word_vectors.T)
    q = (target // FBLK) * HBLK + (target % HBLK)
    k = (target % FBLK) // HBLK
    idx = q.reshape(NUM_WORKERS * CHUNKS_PER_WORKER, CHUNK)
    par = k.astype(jnp.float32)
    sel = _sc_gather(table_packed, idx).reshape(BATCH, WIDE)
    out = _cos_loss(logits.T, sel, par, mask)
    return out[0, 0]


# speedup vs baseline: 2.3949x; 1.0541x over previous
import jax
import jax.numpy as jnp
from jax import lax
from jax.experimental import pallas as pl
from jax.experimental.pallas import tpu as pltpu
from jax.experimental.pallas import tpu_sc as plsc

BATCH = 16384
VOCAB = 1000000
DIM = 64
WIDE = 2 * DIM

NUM_CORES = 2
NUM_SUBCORES = 16
NUM_WORKERS = NUM_CORES * NUM_SUBCORES
CHUNK = 128
CHUNKS_PER_WORKER = BATCH // (NUM_WORKERS * CHUNK)

HBLK = 16384
FBLK = 2 * HBLK
RGRID = -(-VOCAB // FBLK)
PACKED_ROWS = RGRID * HBLK


def _relayout_body(a_ref, b_ref, out_ref):
    out_ref[:, :DIM] = jnp.transpose(a_ref[...])
    out_ref[:, DIM:] = jnp.transpose(b_ref[...])


def _relayout(wv_t):
    return pl.pallas_call(
        _relayout_body,
        grid=(RGRID,),
        in_specs=[
            pl.BlockSpec(
                (DIM, HBLK),
                lambda i: (0, jnp.minimum(2 * i, VOCAB // HBLK)),
            ),
            pl.BlockSpec(
                (DIM, HBLK),
                lambda i: (0, jnp.minimum(2 * i + 1, VOCAB // HBLK)),
            ),
        ],
        out_specs=pl.BlockSpec((HBLK, WIDE), lambda i: (i, 0)),
        out_shape=jax.ShapeDtypeStruct((PACKED_ROWS, WIDE), jnp.float32),
    )(wv_t, wv_t)


def _sc_gather_body(table_hbm, idx_hbm, out_hbm, idx_v, rows_v, sem):
    c = lax.axis_index("c")
    s = lax.axis_index("s")
    wid = s * NUM_CORES + c
    base = wid * CHUNKS_PER_WORKER
    pltpu.sync_copy(idx_hbm.at[pl.ds(base, CHUNKS_PER_WORKER)], idx_v)
    copies = [
        pltpu.async_copy(table_hbm.at[idx_v.at[j]], rows_v.at[j], sem)
        for j in range(CHUNKS_PER_WORKER)
    ]
    for cp in copies:
        cp.wait()
    pltpu.sync_copy(rows_v, out_hbm.at[pl.ds(base, CHUNKS_PER_WORKER)])


def _sc_gather(table_packed, idx):
    mesh = plsc.VectorSubcoreMesh(core_axis_name="c", subcore_axis_name="s")
    kfn = pl.kernel(
        _sc_gather_body,
        out_type=jax.ShapeDtypeStruct(
            (NUM_WORKERS * CHUNKS_PER_WORKER, CHUNK, WIDE), jnp.float32
        ),
        mesh=mesh,
        scratch_types=[
            pltpu.VMEM((CHUNKS_PER_WORKER, CHUNK), jnp.int32),
            pltpu.VMEM((CHUNKS_PER_WORKER, CHUNK, WIDE), jnp.float32),
            pltpu.SemaphoreType.DMA,
        ],
    )
    return kfn(table_packed, idx)


GRID = 8
BLK = BATCH // GRID


def _cos_body(xt_ref, sel_ref, par_ref, mask_ref, out_ref, acc_ref):
    i = pl.program_id(0)

    @pl.when(i == 0)
    def _():
        acc_ref[0] = 0.0
        acc_ref[1] = 0.0

    x = xt_ref[...]
    s2 = sel_ref[...]
    p = par_ref[...]
    m = mask_ref[...]
    eye = jnp.eye(WIDE, dtype=jnp.float32)
    ts = jax.lax.dot_general(
        eye,
        s2,
        dimension_numbers=(((1,), (1,)), ((), ())),
        precision=jax.lax.Precision.HIGHEST,
    )
    s = jnp.where(p[None, :] > 0.5, ts[DIM:, :], ts[:DIM, :])
    num = jnp.sum(x * s, axis=0)
    n1s = jnp.sum(x * x, axis=0)
    n2s = jnp.sum(s * s, axis=0)
    denom = jnp.maximum(jnp.sqrt(n1s) * jnp.sqrt(n2s), 1e-8)
    acc_ref[0] += jnp.sum(-(num / denom) * m)
    acc_ref[1] += jnp.sum(m)

    @pl.when(i == GRID - 1)
    def _():
        out_ref[...] = jnp.full((1, 1), acc_ref[0] / acc_ref[1], jnp.float32)


def _cos_loss(x_t, sel, par, mask):
    return pl.pallas_call(
        _cos_body,
        grid=(GRID,),
        in_specs=[
            pl.BlockSpec((DIM, BLK), lambda i: (0, i)),
            pl.BlockSpec((BLK, WIDE), lambda i: (i, 0)),
            pl.BlockSpec((BLK,), lambda i: (i,)),
            pl.BlockSpec((BLK,), lambda i: (i,)),
        ],
        out_specs=pl.BlockSpec((1, 1), lambda i: (0, 0)),
        out_shape=jax.ShapeDtypeStruct((1, 1), jnp.float32),
        scratch_shapes=[pltpu.SMEM((2,), jnp.float32)],
    )(x_t, sel, par, mask)


def kernel(logits, target, mask, word_vectors):
    table_packed = _relayout(word_vectors.T)
    q = (target // FBLK) * HBLK + (target % HBLK)
    k = (target % FBLK) // HBLK
    idx = q.reshape(NUM_WORKERS * CHUNKS_PER_WORKER, CHUNK)
    par = k.astype(jnp.float32)
    sel = _sc_gather(table_packed, idx).reshape(BATCH, WIDE)
    out = _cos_loss(logits.T, sel, par, mask)
    return out[0, 0]


# speedup vs baseline: 2.4115x; 1.0069x over previous
import jax
import jax.numpy as jnp
from jax import lax
from jax.experimental import pallas as pl
from jax.experimental.pallas import tpu as pltpu
from jax.experimental.pallas import tpu_sc as plsc

BATCH = 16384
VOCAB = 1000000
DIM = 64
WIDE = 2 * DIM

NUM_CORES = 2
NUM_SUBCORES = 16
NUM_WORKERS = NUM_CORES * NUM_SUBCORES
CHUNK = 128
CHUNKS_PER_WORKER = BATCH // (NUM_WORKERS * CHUNK)

HBLK = 16384
FBLK = 2 * HBLK
RGRID = -(-VOCAB // FBLK)
PACKED_ROWS = RGRID * HBLK


def _relayout_body(a_ref, b_ref, out_ref):
    out_ref[:, :DIM] = jnp.transpose(a_ref[...])
    out_ref[:, DIM:] = jnp.transpose(b_ref[...])


def _relayout(wv_t):
    return pl.pallas_call(
        _relayout_body,
        grid=(RGRID,),
        in_specs=[
            pl.BlockSpec(
                (DIM, HBLK),
                lambda i: (0, jnp.minimum(2 * i, VOCAB // HBLK)),
            ),
            pl.BlockSpec(
                (DIM, HBLK),
                lambda i: (0, jnp.minimum(2 * i + 1, VOCAB // HBLK)),
            ),
        ],
        out_specs=pl.BlockSpec((HBLK, WIDE), lambda i: (i, 0)),
        out_shape=jax.ShapeDtypeStruct((PACKED_ROWS, WIDE), jnp.float32),
    )(wv_t, wv_t)


def _sc_gather_body(table_hbm, idx_hbm, out_hbm, idx_v, rows_v, sem):
    c = lax.axis_index("c")
    s = lax.axis_index("s")
    wid = s * NUM_CORES + c
    base = wid * CHUNKS_PER_WORKER
    pltpu.sync_copy(idx_hbm.at[pl.ds(base, CHUNKS_PER_WORKER)], idx_v)
    copies = [
        pltpu.async_copy(table_hbm.at[idx_v.at[j]], rows_v.at[j], sem)
        for j in range(CHUNKS_PER_WORKER)
    ]
    for cp in copies:
        cp.wait()
    pltpu.sync_copy(rows_v, out_hbm.at[pl.ds(base, CHUNKS_PER_WORKER)])


def _sc_gather(table_packed, idx):
    mesh = plsc.VectorSubcoreMesh(core_axis_name="c", subcore_axis_name="s")
    kfn = pl.kernel(
        _sc_gather_body,
        out_type=jax.ShapeDtypeStruct(
            (NUM_WORKERS * CHUNKS_PER_WORKER, CHUNK, WIDE), jnp.float32
        ),
        mesh=mesh,
        scratch_types=[
            pltpu.VMEM((CHUNKS_PER_WORKER, CHUNK), jnp.int32),
            pltpu.VMEM((CHUNKS_PER_WORKER, CHUNK, WIDE), jnp.float32),
            pltpu.SemaphoreType.DMA,
        ],
    )
    return kfn(table_packed, idx)


GRID = 4
BLK = BATCH // GRID


def _cos_body(xt_ref, sel_ref, par_ref, mask_ref, out_ref, acc_ref):
    i = pl.program_id(0)

    @pl.when(i == 0)
    def _():
        acc_ref[0] = 0.0
        acc_ref[1] = 0.0

    x = xt_ref[...]
    s2 = sel_ref[...]
    p = par_ref[...]
    m = mask_ref[...]
    eye = jnp.eye(WIDE, dtype=jnp.float32)
    ts = jax.lax.dot_general(
        eye,
        s2,
        dimension_numbers=(((1,), (1,)), ((), ())),
        precision=jax.lax.Precision.HIGHEST,
    )
    s = jnp.where(p[None, :] > 0.5, ts[DIM:, :], ts[:DIM, :])
    num = jnp.sum(x * s, axis=0)
    n1s = jnp.sum(x * x, axis=0)
    n2s = jnp.sum(s * s, axis=0)
    denom = jnp.maximum(jnp.sqrt(n1s) * jnp.sqrt(n2s), 1e-8)
    acc_ref[0] += jnp.sum(-(num / denom) * m)
    acc_ref[1] += jnp.sum(m)

    @pl.when(i == GRID - 1)
    def _():
        out_ref[...] = jnp.full((1, 1), acc_ref[0] / acc_ref[1], jnp.float32)


def _cos_loss(x_t, sel, par, mask):
    return pl.pallas_call(
        _cos_body,
        grid=(GRID,),
        in_specs=[
            pl.BlockSpec((DIM, BLK), lambda i: (0, i)),
            pl.BlockSpec((BLK, WIDE), lambda i: (i, 0)),
            pl.BlockSpec((BLK,), lambda i: (i,)),
            pl.BlockSpec((BLK,), lambda i: (i,)),
        ],
        out_specs=pl.BlockSpec((1, 1), lambda i: (0, 0)),
        out_shape=jax.ShapeDtypeStruct((1, 1), jnp.float32),
        scratch_shapes=[pltpu.SMEM((2,), jnp.float32)],
    )(x_t, sel, par, mask)


def kernel(logits, target, mask, word_vectors):
    table_packed = _relayout(word_vectors.T)
    q = (target // FBLK) * HBLK + (target % HBLK)
    k = (target % FBLK) // HBLK
    idx = q.reshape(NUM_WORKERS * CHUNKS_PER_WORKER, CHUNK)
    par = k.astype(jnp.float32)
    sel = _sc_gather(table_packed, idx).reshape(BATCH, WIDE)
    out = _cos_loss(logits.T, sel, par, mask)
    return out[0, 0]
